# baseline (speedup 1.0000x reference)
import functools

import numpy as np
import jax
import jax.numpy as jnp
from jax.experimental import pallas as pl
from jax.experimental.pallas import tpu as pltpu

N_PRIORS = 20000
N_CLASSES = 80
THRESHOLD = 0.7
BIG_I32 = np.int32(2**30)


_ATAN_COEFFS = (0.9999993329, -0.3332985605, 0.1994653599, -0.1390853351,
                0.0964200441, -0.0559098861, 0.0218612288, -0.0040454960)


def _arctan(x):
    ax = jnp.abs(x)
    inv = ax > 1.0
    t = jnp.where(inv, 1.0 / jnp.maximum(ax, 1e-30), ax)
    t2 = t * t
    poly = _ATAN_COEFFS[-1]
    for c in _ATAN_COEFFS[-2::-1]:
        poly = poly * t2 + c
    poly = poly * t
    r = jnp.where(inv, np.float32(np.pi / 2) - poly, poly)
    return jnp.where(x < 0, -r, r)


def _assign_kernel(boxes_ref, labels_ref, priors_ref, lab_ref, tlocs_ref):
    boxes = boxes_ref[0]
    bx0 = boxes[:, 0:1]
    by0 = boxes[:, 1:2]
    bx1 = boxes[:, 2:3]
    by1 = boxes[:, 3:4]
    px0 = priors_ref[0:1, :]
    py0 = priors_ref[1:2, :]
    px1 = priors_ref[2:3, :]
    py1 = priors_ref[3:4, :]

    iw = jnp.clip(jnp.minimum(bx1, px1) - jnp.maximum(bx0, px0), 0.0, None)
    ih = jnp.clip(jnp.minimum(by1, py1) - jnp.maximum(by0, py0), 0.0, None)
    inter = iw * ih
    a1 = (bx1 - bx0) * (by1 - by0)
    a2 = (px1 - px0) * (py1 - py0)
    overlap = inter / (a1 + a2 - inter)

    n_obj = overlap.shape[0]
    o_iota = jax.lax.broadcasted_iota(jnp.int32, (n_obj, 1), 0)
    p_iota = jax.lax.broadcasted_iota(jnp.int32, (1, overlap.shape[1]), 1)

    ofp = jnp.max(overlap, axis=0, keepdims=True)
    obj_fp = jnp.min(
        jnp.where(overlap == ofp, o_iota, BIG_I32), axis=0, keepdims=True
    )

    ofo = jnp.max(overlap, axis=1, keepdims=True)
    pfo = jnp.min(
        jnp.where(overlap == ofo, p_iota, BIG_I32), axis=1, keepdims=True
    )
    valid = ofo > 0.0

    match = (pfo == p_iota) & valid
    cand = jnp.where(match, o_iota, np.int32(-1))
    forced = jnp.max(cand, axis=0, keepdims=True)
    has = forced >= 0
    obj2 = jnp.where(has, forced, obj_fp)
    ofp2 = jnp.where(has, 1.0, ofp)

    onehot = (obj2 == o_iota).astype(jnp.float32)
    vals = jnp.concatenate([labels_ref[0], bx0, by0, bx1, by1], axis=1)
    gathered = jax.lax.dot_general(
        vals, onehot, (((0,), (0,)), ((), ())),
        preferred_element_type=jnp.float32)
    lab_raw = gathered[0:1, :]
    lab = jnp.where(ofp2 < THRESHOLD, -1.0, lab_raw)
    lab = jnp.where(ofp2 < THRESHOLD - 0.25, 0.0, lab)
    lab_ref[0] = lab
    tlocs_ref[0] = gathered[1:5, :]


def _loss_kernel(locs_ref, pri_ref, scores_ref, lab_row_ref, lab_col_ref,
                 tlocs_ref, loc_ref, npos_ref, focal_ref):
    j = pl.program_id(1)

    @pl.when(j == 0)
    def _():
        loc_ref[...] = jnp.zeros_like(loc_ref)
        npos_ref[...] = jnp.zeros_like(npos_ref)
        focal_ref[...] = jnp.zeros_like(focal_ref)

    @pl.when(j == 0)
    def _():
        locs = locs_ref[0]
        pri = pri_ref[...]
        cx = locs[0:1, :] * pri[2:3, :] / 10.0 + pri[0:1, :]
        cy = locs[1:2, :] * pri[3:4, :] / 10.0 + pri[1:2, :]
        w1 = jnp.exp(locs[2:3, :] / 5.0) * pri[2:3, :]
        h1 = jnp.exp(locs[3:4, :] / 5.0) * pri[3:4, :]
        b1x0 = cx - w1 / 2.0
        b1y0 = cy - h1 / 2.0
        b1x1 = cx + w1 / 2.0
        b1y1 = cy + h1 / 2.0

        tl = tlocs_ref[0]
        b2x0 = tl[0:1, :]
        b2y0 = tl[1:2, :]
        b2x1 = tl[2:3, :]
        b2y1 = tl[3:4, :]
        w2 = b2x1 - b2x0
        h2 = b2y1 - b2y0

        iw = jnp.clip(jnp.minimum(b1x1, b2x1) - jnp.maximum(b1x0, b2x0), 0.0, None)
        ih = jnp.clip(jnp.minimum(b1y1, b2y1) - jnp.maximum(b1y0, b2y0), 0.0, None)
        inter = iw * ih
        union = w1 * h1 + w2 * h2 - inter
        iou = inter / (union + 1e-7)
        ew = jnp.clip(jnp.maximum(b1x1, b2x1) - jnp.minimum(b1x0, b2x0), 0.0, None)
        eh = jnp.clip(jnp.maximum(b1y1, b2y1) - jnp.minimum(b1y0, b2y0), 0.0, None)
        cdiag = ew * ew + eh * eh + 1e-7
        c2x = (b2x0 + b2x1) / 2.0
        c2y = (b2y0 + b2y1) / 2.0
        rho2 = (cx - c2x) ** 2 + (cy - c2y) ** 2
        u = rho2 / cdiag
        v = (4.0 / (np.pi ** 2)) * (
            _arctan(w2 / (h2 + 1e-7)) - _arctan(w1 / (h1 + 1e-7))
        ) ** 2
        alpha = v / ((1.0 - iou) + v + 1e-7)
        ciou = iou - (u + alpha * v)

        lab_row = lab_row_ref[0]
        posf = (lab_row > 0.0).astype(jnp.float32)
        loc_ref[0] += jnp.sum(posf * (1.0 - ciou), keepdims=True)
        npos_ref[0] += jnp.sum(posf, keepdims=True)

    x = scores_ref[0]
    t = lab_col_ref[0]
    cr = jax.lax.broadcasted_iota(jnp.int32, (1, x.shape[1]), 1).astype(
        jnp.float32) + 1.0
    u = jnp.exp(-x)
    w = 1.0 + u
    r = 1.0 / w
    s = jnp.log(w)
    t2n = (r * r) * (x + s)
    xsel = jnp.where(t == cr, x, 0.0)
    ones_c = jnp.ones((x.shape[1], 1), dtype=jnp.float32)
    s2 = jax.lax.dot_general(
        t2n, ones_c, (((1,), (0,)), ((), ())),
        preferred_element_type=jnp.float32)
    xs = jax.lax.dot_general(
        xsel, ones_c, (((1,), (0,)), ((), ())),
        preferred_element_type=jnp.float32)
    row_ok = (t >= 0.0).astype(jnp.float32)
    neg_all = jnp.sum(row_ok * s2, keepdims=True)
    us = jnp.exp(-xs)
    ps = 1.0 / (1.0 + us)
    ss = jnp.log(1.0 + us)
    t1n_s = (us * ps) ** 2 * ss
    t2n_s = ps * ps * (xs + ss)
    pos_row = (t > 0.0).astype(jnp.float32)
    corr = jnp.sum(pos_row * (0.25 * t1n_s - 0.75 * t2n_s), keepdims=True)
    focal_ref[0] += 0.75 * neg_all + corr


@functools.partial(jax.jit, static_argnames=("interpret",))
def _run(odm_locs, odm_scores, boxes, labels, priors_cxcy, interpret=False):
    B, P, C = odm_scores.shape
    O = boxes.shape[1]
    priors_cxcy_t = priors_cxcy.T
    priors_xy_t = jnp.concatenate(
        [priors_cxcy_t[:2] - priors_cxcy_t[2:] / 2.0,
         priors_cxcy_t[:2] + priors_cxcy_t[2:] / 2.0], axis=0)
    labels_col = labels.astype(jnp.float32).reshape(B, O, 1)

    lab_row, tlocs_t = pl.pallas_call(
        _assign_kernel,
        grid=(B,),
        in_specs=[
            pl.BlockSpec((1, O, 4), lambda i: (i, 0, 0)),
            pl.BlockSpec((1, O, 1), lambda i: (i, 0, 0)),
            pl.BlockSpec((4, P), lambda i: (0, 0)),
        ],
        out_specs=[
            pl.BlockSpec((1, 1, P), lambda i: (i, 0, 0)),
            pl.BlockSpec((1, 4, P), lambda i: (i, 0, 0)),
        ],
        out_shape=[
            jax.ShapeDtypeStruct((B, 1, P), jnp.float32),
            jax.ShapeDtypeStruct((B, 4, P), jnp.float32),
        ],
        compiler_params=pltpu.CompilerParams(
            dimension_semantics=("parallel",)),
        interpret=interpret,
    )(boxes, labels_col, priors_xy_t)

    lab_col = lab_row.reshape(B, P, 1)
    locs_t = jnp.transpose(odm_locs, (0, 2, 1))

    T = 10000
    NT = P // T
    loc_s, npos, focal_s = pl.pallas_call(
        _loss_kernel,
        grid=(B, NT),
        in_specs=[
            pl.BlockSpec((1, 4, P), lambda i, j: (i, 0, 0)),
            pl.BlockSpec((4, P), lambda i, j: (0, 0)),
            pl.BlockSpec((1, T, C), lambda i, j: (i, j, 0)),
            pl.BlockSpec((1, 1, P), lambda i, j: (i, 0, 0)),
            pl.BlockSpec((1, T, 1), lambda i, j: (i, j, 0)),
            pl.BlockSpec((1, 4, P), lambda i, j: (i, 0, 0)),
        ],
        out_specs=[
            pl.BlockSpec((1, 1, 1), lambda i, j: (i, 0, 0)),
            pl.BlockSpec((1, 1, 1), lambda i, j: (i, 0, 0)),
            pl.BlockSpec((1, 1, 1), lambda i, j: (i, 0, 0)),
        ],
        out_shape=[
            jax.ShapeDtypeStruct((B, 1, 1), jnp.float32),
            jax.ShapeDtypeStruct((B, 1, 1), jnp.float32),
            jax.ShapeDtypeStruct((B, 1, 1), jnp.float32),
        ],
        compiler_params=pltpu.CompilerParams(
            dimension_semantics=("parallel", "arbitrary")),
        interpret=interpret,
    )(locs_t, priors_cxcy_t, odm_scores, lab_row, lab_col, tlocs_t)

    n_pos = jnp.sum(npos)
    return (jnp.sum(focal_s) + jnp.sum(loc_s)) / n_pos


def kernel(odm_locs, odm_scores, boxes, labels, priors_cxcy):
    return _run(odm_locs, odm_scores, boxes, labels, priors_cxcy)


# speedup vs baseline: 1.0445x; 1.0445x over previous
import functools

import numpy as np
import jax
import jax.numpy as jnp
from jax.experimental import pallas as pl
from jax.experimental.pallas import tpu as pltpu

N_PRIORS = 20000
N_CLASSES = 80
THRESHOLD = 0.7
BIG_I32 = np.int32(2**30)

_ATAN_COEFFS = (0.9999993329, -0.3332985605, 0.1994653599, -0.1390853351,
                0.0964200441, -0.0559098861, 0.0218612288, -0.0040454960)


def _arctan(x):
    ax = jnp.abs(x)
    inv = ax > 1.0
    t = jnp.where(inv, 1.0 / jnp.maximum(ax, 1e-30), ax)
    t2 = t * t
    poly = _ATAN_COEFFS[-1]
    for c in _ATAN_COEFFS[-2::-1]:
        poly = poly * t2 + c
    poly = poly * t
    r = jnp.where(inv, np.float32(np.pi / 2) - poly, poly)
    return jnp.where(x < 0, -r, r)


def _fused_kernel(boxes_ref, labels_ref, pxy_ref, pcxcy_ref, locs_ref,
                  scores_ref, loc_ref, npos_ref, focal_ref, labcol_ref):
    j = pl.program_id(1)
    T = scores_ref.shape[1]

    @pl.when(j == 0)
    def _():
        loc_ref[...] = jnp.zeros_like(loc_ref)
        npos_ref[...] = jnp.zeros_like(npos_ref)
        focal_ref[...] = jnp.zeros_like(focal_ref)

        boxes = boxes_ref[0]
        bx0 = boxes[:, 0:1]
        by0 = boxes[:, 1:2]
        bx1 = boxes[:, 2:3]
        by1 = boxes[:, 3:4]
        px0 = pxy_ref[0:1, :]
        py0 = pxy_ref[1:2, :]
        px1 = pxy_ref[2:3, :]
        py1 = pxy_ref[3:4, :]

        iw0 = jnp.clip(jnp.minimum(bx1, px1) - jnp.maximum(bx0, px0), 0.0, None)
        ih0 = jnp.clip(jnp.minimum(by1, py1) - jnp.maximum(by0, py0), 0.0, None)
        inter0 = iw0 * ih0
        a1 = (bx1 - bx0) * (by1 - by0)
        a2 = (px1 - px0) * (py1 - py0)
        overlap = inter0 / (a1 + a2 - inter0)

        n_obj = overlap.shape[0]
        o_iota = jax.lax.broadcasted_iota(jnp.int32, (n_obj, 1), 0)
        p_iota = jax.lax.broadcasted_iota(jnp.int32, (1, overlap.shape[1]), 1)

        ofp = jnp.max(overlap, axis=0, keepdims=True)
        obj_fp = jnp.min(
            jnp.where(overlap == ofp, o_iota, BIG_I32), axis=0, keepdims=True)

        ofo = jnp.max(overlap, axis=1, keepdims=True)
        pfo = jnp.min(
            jnp.where(overlap == ofo, p_iota, BIG_I32), axis=1, keepdims=True)
        valid = ofo > 0.0

        match = (pfo == p_iota) & valid
        cand = jnp.where(match, o_iota, np.int32(-1))
        forced = jnp.max(cand, axis=0, keepdims=True)
        has = forced >= 0
        obj2 = jnp.where(has, forced, obj_fp)
        ofp2 = jnp.where(has, 1.0, ofp)

        onehot = (obj2 == o_iota).astype(jnp.float32)
        vals = jnp.concatenate([labels_ref[0], bx0, by0, bx1, by1], axis=1)
        gathered = jax.lax.dot_general(
            vals, onehot, (((0,), (0,)), ((), ())),
            preferred_element_type=jnp.float32)
        lab = jnp.where(ofp2 < THRESHOLD, -1.0, gathered[0:1, :])
        lab = jnp.where(ofp2 < THRESHOLD - 0.25, 0.0, lab)
        labcol_ref[...] = lab.reshape(labcol_ref.shape)

        locs = locs_ref[0]
        pri = pcxcy_ref[...]
        cx = locs[0:1, :] * pri[2:3, :] / 10.0 + pri[0:1, :]
        cy = locs[1:2, :] * pri[3:4, :] / 10.0 + pri[1:2, :]
        w1 = jnp.exp(locs[2:3, :] / 5.0) * pri[2:3, :]
        h1 = jnp.exp(locs[3:4, :] / 5.0) * pri[3:4, :]
        b1x0 = cx - w1 / 2.0
        b1y0 = cy - h1 / 2.0
        b1x1 = cx + w1 / 2.0
        b1y1 = cy + h1 / 2.0

        b2x0 = gathered[1:2, :]
        b2y0 = gathered[2:3, :]
        b2x1 = gathered[3:4, :]
        b2y1 = gathered[4:5, :]
        w2 = b2x1 - b2x0
        h2 = b2y1 - b2y0

        iw = jnp.clip(jnp.minimum(b1x1, b2x1) - jnp.maximum(b1x0, b2x0),
                      0.0, None)
        ih = jnp.clip(jnp.minimum(b1y1, b2y1) - jnp.maximum(b1y0, b2y0),
                      0.0, None)
        inter = iw * ih
        union = w1 * h1 + w2 * h2 - inter
        iou = inter / (union + 1e-7)
        ew = jnp.clip(jnp.maximum(b1x1, b2x1) - jnp.minimum(b1x0, b2x0),
                      0.0, None)
        eh = jnp.clip(jnp.maximum(b1y1, b2y1) - jnp.minimum(b1y0, b2y0),
                      0.0, None)
        cdiag = ew * ew + eh * eh + 1e-7
        c2x = (b2x0 + b2x1) / 2.0
        c2y = (b2y0 + b2y1) / 2.0
        rho2 = (cx - c2x) ** 2 + (cy - c2y) ** 2
        u = rho2 / cdiag
        v = (4.0 / (np.pi ** 2)) * (
            _arctan(w2 / (h2 + 1e-7)) - _arctan(w1 / (h1 + 1e-7))
        ) ** 2
        alpha = v / ((1.0 - iou) + v + 1e-7)
        ciou = iou - (u + alpha * v)

        posf = (lab > 0.0).astype(jnp.float32)
        loc_ref[0] += jnp.sum(posf * (1.0 - ciou), keepdims=True)
        npos_ref[0] += jnp.sum(posf, keepdims=True)

    x = scores_ref[0]
    t = labcol_ref[pl.ds(j * T, T), :]
    cr = jax.lax.broadcasted_iota(jnp.int32, (1, x.shape[1]), 1).astype(
        jnp.float32) + 1.0
    u = jnp.exp(-x)
    w = 1.0 + u
    r = 1.0 / w
    s = jnp.log(w)
    t2n = (r * r) * (x + s)
    xsel = jnp.where(t == cr, x, 0.0)
    ones_c = jnp.ones((x.shape[1], 1), dtype=jnp.float32)
    s2 = jax.lax.dot_general(
        t2n, ones_c, (((1,), (0,)), ((), ())),
        preferred_element_type=jnp.float32)
    xs = jax.lax.dot_general(
        xsel, ones_c, (((1,), (0,)), ((), ())),
        preferred_element_type=jnp.float32)
    row_ok = (t >= 0.0).astype(jnp.float32)
    neg_all = jnp.sum(row_ok * s2, keepdims=True)
    us = jnp.exp(-xs)
    ps = 1.0 / (1.0 + us)
    ss = jnp.log(1.0 + us)
    t1n_s = (us * ps) ** 2 * ss
    t2n_s = ps * ps * (xs + ss)
    pos_row = (t > 0.0).astype(jnp.float32)
    corr = jnp.sum(pos_row * (0.25 * t1n_s - 0.75 * t2n_s), keepdims=True)
    focal_ref[0] += 0.75 * neg_all + corr


@functools.partial(jax.jit, static_argnames=("interpret",))
def _run(odm_locs, odm_scores, boxes, labels, priors_cxcy, interpret=False):
    B, P, C = odm_scores.shape
    O = boxes.shape[1]
    priors_cxcy_t = priors_cxcy.T
    priors_xy_t = jnp.concatenate(
        [priors_cxcy_t[:2] - priors_cxcy_t[2:] / 2.0,
         priors_cxcy_t[:2] + priors_cxcy_t[2:] / 2.0], axis=0)
    labels_col = labels.astype(jnp.float32).reshape(B, O, 1)
    locs_t = jnp.transpose(odm_locs, (0, 2, 1))

    T = 5000
    NT = P // T
    loc_s, npos, focal_s = pl.pallas_call(
        _fused_kernel,
        grid=(B, NT),
        in_specs=[
            pl.BlockSpec((1, O, 4), lambda i, j: (i, 0, 0)),
            pl.BlockSpec((1, O, 1), lambda i, j: (i, 0, 0)),
            pl.BlockSpec((4, P), lambda i, j: (0, 0)),
            pl.BlockSpec((4, P), lambda i, j: (0, 0)),
            pl.BlockSpec((1, 4, P), lambda i, j: (i, 0, 0)),
            pl.BlockSpec((1, T, C), lambda i, j: (i, j, 0)),
        ],
        out_specs=[
            pl.BlockSpec((1, 1, 1), lambda i, j: (i, 0, 0)),
            pl.BlockSpec((1, 1, 1), lambda i, j: (i, 0, 0)),
            pl.BlockSpec((1, 1, 1), lambda i, j: (i, 0, 0)),
        ],
        out_shape=[
            jax.ShapeDtypeStruct((B, 1, 1), jnp.float32),
            jax.ShapeDtypeStruct((B, 1, 1), jnp.float32),
            jax.ShapeDtypeStruct((B, 1, 1), jnp.float32),
        ],
        scratch_shapes=[pltpu.VMEM((P, 1), jnp.float32)],
        interpret=interpret,
    )(boxes, labels_col, priors_xy_t, priors_cxcy_t, locs_t, odm_scores)

    n_pos = jnp.sum(npos)
    return (jnp.sum(focal_s) + jnp.sum(loc_s)) / n_pos


def kernel(odm_locs, odm_scores, boxes, labels, priors_cxcy):
    return _run(odm_locs, odm_scores, boxes, labels, priors_cxcy)


# speedup vs baseline: 1.1390x; 1.0904x over previous
import functools

import numpy as np
import jax
import jax.numpy as jnp
from jax.experimental import pallas as pl
from jax.experimental.pallas import tpu as pltpu

N_PRIORS = 20000
N_CLASSES = 80
THRESHOLD = 0.7
BIG_I32 = np.int32(2**30)

_ATAN_COEFFS = (0.9999993329, -0.3332985605, 0.1994653599, -0.1390853351,
                0.0964200441, -0.0559098861, 0.0218612288, -0.0040454960)


def _arctan(x):
    ax = jnp.abs(x)
    inv = ax > 1.0
    t = jnp.where(inv, 1.0 / jnp.maximum(ax, 1e-30), ax)
    t2 = t * t
    poly = _ATAN_COEFFS[-1]
    for c in _ATAN_COEFFS[-2::-1]:
        poly = poly * t2 + c
    poly = poly * t
    r = jnp.where(inv, np.float32(np.pi / 2) - poly, poly)
    return jnp.where(x < 0, -r, r)


def _fused_kernel(boxes_ref, labels_ref, pxy_ref, pcxcy_ref, locs_ref,
                  scores_ref, loc_ref, npos_ref, focal_ref, labcol_ref):
    j = pl.program_id(1)
    T = scores_ref.shape[1]

    @pl.when(j == 0)
    def _():
        loc_ref[...] = jnp.zeros_like(loc_ref)
        npos_ref[...] = jnp.zeros_like(npos_ref)
        focal_ref[...] = jnp.zeros_like(focal_ref)

        boxes = boxes_ref[0]
        bx0 = boxes[:, 0:1]
        by0 = boxes[:, 1:2]
        bx1 = boxes[:, 2:3]
        by1 = boxes[:, 3:4]
        px0 = pxy_ref[0:1, :]
        py0 = pxy_ref[1:2, :]
        px1 = pxy_ref[2:3, :]
        py1 = pxy_ref[3:4, :]

        iw0 = jnp.clip(jnp.minimum(bx1, px1) - jnp.maximum(bx0, px0), 0.0, None)
        ih0 = jnp.clip(jnp.minimum(by1, py1) - jnp.maximum(by0, py0), 0.0, None)
        inter0 = iw0 * ih0
        a1 = (bx1 - bx0) * (by1 - by0)
        a2 = (px1 - px0) * (py1 - py0)
        overlap = inter0 / (a1 + a2 - inter0)

        n_obj = overlap.shape[0]
        o_iota = jax.lax.broadcasted_iota(jnp.int32, (n_obj, 1), 0)
        p_iota = jax.lax.broadcasted_iota(jnp.int32, (1, overlap.shape[1]), 1)

        ofp = jnp.max(overlap, axis=0, keepdims=True)
        obj_fp = jnp.min(
            jnp.where(overlap == ofp, o_iota, BIG_I32), axis=0, keepdims=True)

        ofo = jnp.max(overlap, axis=1, keepdims=True)
        pfo = jnp.min(
            jnp.where(overlap == ofo, p_iota, BIG_I32), axis=1, keepdims=True)
        valid = ofo > 0.0

        match = (pfo == p_iota) & valid
        cand = jnp.where(match, o_iota, np.int32(-1))
        forced = jnp.max(cand, axis=0, keepdims=True)
        has = forced >= 0
        obj2 = jnp.where(has, forced, obj_fp)
        ofp2 = jnp.where(has, 1.0, ofp)

        onehot = (obj2 == o_iota).astype(jnp.float32)
        vals = jnp.concatenate([labels_ref[0], bx0, by0, bx1, by1], axis=1)
        gathered = jax.lax.dot_general(
            vals, onehot, (((0,), (0,)), ((), ())),
            preferred_element_type=jnp.float32)
        lab = jnp.where(ofp2 < THRESHOLD, -1.0, gathered[0:1, :])
        lab = jnp.where(ofp2 < THRESHOLD - 0.25, 0.0, lab)
        labcol_ref[...] = lab.reshape(labcol_ref.shape)

        locs = locs_ref[0]
        pri = pcxcy_ref[...]
        cx = locs[0:1, :] * pri[2:3, :] / 10.0 + pri[0:1, :]
        cy = locs[1:2, :] * pri[3:4, :] / 10.0 + pri[1:2, :]
        w1 = jnp.exp(locs[2:3, :] / 5.0) * pri[2:3, :]
        h1 = jnp.exp(locs[3:4, :] / 5.0) * pri[3:4, :]
        b1x0 = cx - w1 / 2.0
        b1y0 = cy - h1 / 2.0
        b1x1 = cx + w1 / 2.0
        b1y1 = cy + h1 / 2.0

        b2x0 = gathered[1:2, :]
        b2y0 = gathered[2:3, :]
        b2x1 = gathered[3:4, :]
        b2y1 = gathered[4:5, :]
        w2 = b2x1 - b2x0
        h2 = b2y1 - b2y0

        iw = jnp.clip(jnp.minimum(b1x1, b2x1) - jnp.maximum(b1x0, b2x0),
                      0.0, None)
        ih = jnp.clip(jnp.minimum(b1y1, b2y1) - jnp.maximum(b1y0, b2y0),
                      0.0, None)
        inter = iw * ih
        union = w1 * h1 + w2 * h2 - inter
        iou = inter / (union + 1e-7)
        ew = jnp.clip(jnp.maximum(b1x1, b2x1) - jnp.minimum(b1x0, b2x0),
                      0.0, None)
        eh = jnp.clip(jnp.maximum(b1y1, b2y1) - jnp.minimum(b1y0, b2y0),
                      0.0, None)
        cdiag = ew * ew + eh * eh + 1e-7
        c2x = (b2x0 + b2x1) / 2.0
        c2y = (b2y0 + b2y1) / 2.0
        rho2 = (cx - c2x) ** 2 + (cy - c2y) ** 2
        u = rho2 / cdiag
        v = (4.0 / (np.pi ** 2)) * (
            _arctan(w2 / (h2 + 1e-7)) - _arctan(w1 / (h1 + 1e-7))
        ) ** 2
        alpha = v / ((1.0 - iou) + v + 1e-7)
        ciou = iou - (u + alpha * v)

        posf = (lab > 0.0).astype(jnp.float32)
        loc_ref[0] += jnp.sum(posf * (1.0 - ciou), keepdims=True)
        npos_ref[0] += jnp.sum(posf, keepdims=True)

    x = scores_ref[0]
    t = labcol_ref[pl.ds(j * T, T), :]
    cr = jax.lax.broadcasted_iota(jnp.int32, (1, x.shape[1]), 1).astype(
        jnp.float32) + 1.0
    u = jnp.exp(-x)
    w = 1.0 + u
    r = 1.0 / w
    s = jnp.log(w)
    t2n = (r * r) * (x + s)
    xsel = jnp.where(t == cr, x, 0.0)
    ones_c = jnp.ones((x.shape[1], 1), dtype=jnp.float32)
    s2 = jax.lax.dot_general(
        t2n, ones_c, (((1,), (0,)), ((), ())),
        preferred_element_type=jnp.float32)
    xs = jax.lax.dot_general(
        xsel, ones_c, (((1,), (0,)), ((), ())),
        preferred_element_type=jnp.float32)
    row_ok = (t >= 0.0).astype(jnp.float32)
    neg_all = jnp.sum(row_ok * s2, keepdims=True)
    us = jnp.exp(-xs)
    ps = 1.0 / (1.0 + us)
    ss = jnp.log(1.0 + us)
    t1n_s = (us * ps) ** 2 * ss
    t2n_s = ps * ps * (xs + ss)
    pos_row = (t > 0.0).astype(jnp.float32)
    corr = jnp.sum(pos_row * (0.25 * t1n_s - 0.75 * t2n_s), keepdims=True)
    focal_ref[0] += 0.75 * neg_all + corr


@functools.partial(jax.jit, static_argnames=("interpret",))
def _run(odm_locs, odm_scores, boxes, labels, priors_cxcy, interpret=False):
    B, P, C = odm_scores.shape
    O = boxes.shape[1]
    priors_cxcy_t = priors_cxcy.T
    priors_xy_t = jnp.concatenate(
        [priors_cxcy_t[:2] - priors_cxcy_t[2:] / 2.0,
         priors_cxcy_t[:2] + priors_cxcy_t[2:] / 2.0], axis=0)
    labels_col = labels.astype(jnp.float32).reshape(B, O, 1)
    locs_t = jnp.transpose(odm_locs, (0, 2, 1))

    T = 20000
    NT = P // T
    loc_s, npos, focal_s = pl.pallas_call(
        _fused_kernel,
        grid=(B, NT),
        in_specs=[
            pl.BlockSpec((1, O, 4), lambda i, j: (i, 0, 0)),
            pl.BlockSpec((1, O, 1), lambda i, j: (i, 0, 0)),
            pl.BlockSpec((4, P), lambda i, j: (0, 0)),
            pl.BlockSpec((4, P), lambda i, j: (0, 0)),
            pl.BlockSpec((1, 4, P), lambda i, j: (i, 0, 0)),
            pl.BlockSpec((1, T, C), lambda i, j: (i, j, 0)),
        ],
        out_specs=[
            pl.BlockSpec((1, 1, 1), lambda i, j: (i, 0, 0)),
            pl.BlockSpec((1, 1, 1), lambda i, j: (i, 0, 0)),
            pl.BlockSpec((1, 1, 1), lambda i, j: (i, 0, 0)),
        ],
        out_shape=[
            jax.ShapeDtypeStruct((B, 1, 1), jnp.float32),
            jax.ShapeDtypeStruct((B, 1, 1), jnp.float32),
            jax.ShapeDtypeStruct((B, 1, 1), jnp.float32),
        ],
        scratch_shapes=[pltpu.VMEM((P, 1), jnp.float32)],
        interpret=interpret,
    )(boxes, labels_col, priors_xy_t, priors_cxcy_t, locs_t, odm_scores)

    n_pos = jnp.sum(npos)
    return (jnp.sum(focal_s) + jnp.sum(loc_s)) / n_pos


def kernel(odm_locs, odm_scores, boxes, labels, priors_cxcy):
    return _run(odm_locs, odm_scores, boxes, labels, priors_cxcy)


# speedup vs baseline: 1.1480x; 1.0079x over previous
import functools

import numpy as np
import jax
import jax.numpy as jnp
from jax.experimental import pallas as pl

N_PRIORS = 20000
N_CLASSES = 80
THRESHOLD = 0.7
BIG_I32 = np.int32(2**30)

_ATAN_COEFFS = (0.9999993329, -0.3332985605, 0.1994653599, -0.1390853351,
                0.0964200441, -0.0559098861, 0.0218612288, -0.0040454960)


def _arctan(x):
    ax = jnp.abs(x)
    inv = ax > 1.0
    t = jnp.where(inv, 1.0 / jnp.maximum(ax, 1e-30), ax)
    t2 = t * t
    poly = _ATAN_COEFFS[-1]
    for c in _ATAN_COEFFS[-2::-1]:
        poly = poly * t2 + c
    poly = poly * t
    r = jnp.where(inv, np.float32(np.pi / 2) - poly, poly)
    return jnp.where(x < 0, -r, r)


def _fused_kernel(boxes_ref, labels_ref, pxy_ref, pcxcy_ref, locs_ref,
                  scores_ref, loc_ref, npos_ref, focal_ref):
    boxes = boxes_ref[0]
    bx0 = boxes[:, 0:1]
    by0 = boxes[:, 1:2]
    bx1 = boxes[:, 2:3]
    by1 = boxes[:, 3:4]
    px0 = pxy_ref[0:1, :]
    py0 = pxy_ref[1:2, :]
    px1 = pxy_ref[2:3, :]
    py1 = pxy_ref[3:4, :]

    iw0 = jnp.clip(jnp.minimum(bx1, px1) - jnp.maximum(bx0, px0), 0.0, None)
    ih0 = jnp.clip(jnp.minimum(by1, py1) - jnp.maximum(by0, py0), 0.0, None)
    inter0 = iw0 * ih0
    a1 = (bx1 - bx0) * (by1 - by0)
    a2 = (px1 - px0) * (py1 - py0)
    overlap = inter0 / (a1 + a2 - inter0)

    n_obj = overlap.shape[0]
    o_iota = jax.lax.broadcasted_iota(jnp.int32, (n_obj, 1), 0)
    p_iota = jax.lax.broadcasted_iota(jnp.int32, (1, overlap.shape[1]), 1)

    ofp = jnp.max(overlap, axis=0, keepdims=True)
    obj_fp = jnp.min(
        jnp.where(overlap == ofp, o_iota, BIG_I32), axis=0, keepdims=True)

    ofo = jnp.max(overlap, axis=1, keepdims=True)
    pfo = jnp.min(
        jnp.where(overlap == ofo, p_iota, BIG_I32), axis=1, keepdims=True)
    valid = ofo > 0.0

    match = (pfo == p_iota) & valid
    cand = jnp.where(match, o_iota, np.int32(-1))
    forced = jnp.max(cand, axis=0, keepdims=True)
    has = forced >= 0
    obj2 = jnp.where(has, forced, obj_fp)
    ofp2 = jnp.where(has, 1.0, ofp)

    onehot = (obj2 == o_iota).astype(jnp.float32)
    vals = jnp.concatenate([labels_ref[0], bx0, by0, bx1, by1], axis=1)
    gathered = jax.lax.dot_general(
        vals, onehot, (((0,), (0,)), ((), ())),
        preferred_element_type=jnp.float32)
    lab = jnp.where(ofp2 < THRESHOLD, -1.0, gathered[0:1, :])
    lab = jnp.where(ofp2 < THRESHOLD - 0.25, 0.0, lab)

    locs = locs_ref[0]
    pri = pcxcy_ref[...]
    cx = locs[0:1, :] * pri[2:3, :] / 10.0 + pri[0:1, :]
    cy = locs[1:2, :] * pri[3:4, :] / 10.0 + pri[1:2, :]
    w1 = jnp.exp(locs[2:3, :] / 5.0) * pri[2:3, :]
    h1 = jnp.exp(locs[3:4, :] / 5.0) * pri[3:4, :]
    b1x0 = cx - w1 / 2.0
    b1y0 = cy - h1 / 2.0
    b1x1 = cx + w1 / 2.0
    b1y1 = cy + h1 / 2.0

    b2x0 = gathered[1:2, :]
    b2y0 = gathered[2:3, :]
    b2x1 = gathered[3:4, :]
    b2y1 = gathered[4:5, :]
    w2 = b2x1 - b2x0
    h2 = b2y1 - b2y0

    iw = jnp.clip(jnp.minimum(b1x1, b2x1) - jnp.maximum(b1x0, b2x0),
                  0.0, None)
    ih = jnp.clip(jnp.minimum(b1y1, b2y1) - jnp.maximum(b1y0, b2y0),
                  0.0, None)
    inter = iw * ih
    union = w1 * h1 + w2 * h2 - inter
    iou = inter / (union + 1e-7)
    ew = jnp.clip(jnp.maximum(b1x1, b2x1) - jnp.minimum(b1x0, b2x0),
                  0.0, None)
    eh = jnp.clip(jnp.maximum(b1y1, b2y1) - jnp.minimum(b1y0, b2y0),
                  0.0, None)
    cdiag = ew * ew + eh * eh + 1e-7
    c2x = (b2x0 + b2x1) / 2.0
    c2y = (b2y0 + b2y1) / 2.0
    rho2 = (cx - c2x) ** 2 + (cy - c2y) ** 2
    u = rho2 / cdiag
    v = (4.0 / (np.pi ** 2)) * (
        _arctan(w2 / (h2 + 1e-7)) - _arctan(w1 / (h1 + 1e-7))
    ) ** 2
    alpha = v / ((1.0 - iou) + v + 1e-7)
    ciou = iou - (u + alpha * v)

    posf = (lab > 0.0).astype(jnp.float32)
    loc_ref[0] = jnp.sum(posf * (1.0 - ciou), keepdims=True)
    npos_ref[0] = jnp.sum(posf, keepdims=True)

    x = scores_ref[0]
    t = lab.reshape((x.shape[0], 1))
    cr = jax.lax.broadcasted_iota(jnp.int32, (1, x.shape[1]), 1).astype(
        jnp.float32) + 1.0
    un = jnp.exp(-x)
    w = 1.0 + un
    r = 1.0 / w
    s = jnp.log(w)
    t2n = (r * r) * (x + s)
    xsel = jnp.where(t == cr, x, 0.0)
    ones_c = jnp.ones((x.shape[1], 1), dtype=jnp.float32)
    s2 = jax.lax.dot_general(
        t2n, ones_c, (((1,), (0,)), ((), ())),
        preferred_element_type=jnp.float32)
    xs = jax.lax.dot_general(
        xsel, ones_c, (((1,), (0,)), ((), ())),
        preferred_element_type=jnp.float32)
    row_ok = (t >= 0.0).astype(jnp.float32)
    neg_all = jnp.sum(row_ok * s2, keepdims=True)
    us = jnp.exp(-xs)
    ps = 1.0 / (1.0 + us)
    ss = jnp.log(1.0 + us)
    t1n_s = (us * ps) ** 2 * ss
    t2n_s = ps * ps * (xs + ss)
    pos_row = (t > 0.0).astype(jnp.float32)
    corr = jnp.sum(pos_row * (0.25 * t1n_s - 0.75 * t2n_s), keepdims=True)
    focal_ref[0] = 0.75 * neg_all + corr


@functools.partial(jax.jit, static_argnames=("interpret",))
def _run(odm_locs, odm_scores, boxes, labels, priors_cxcy, interpret=False):
    B, P, C = odm_scores.shape
    O = boxes.shape[1]
    priors_cxcy_t = priors_cxcy.T
    priors_xy_t = jnp.concatenate(
        [priors_cxcy_t[:2] - priors_cxcy_t[2:] / 2.0,
         priors_cxcy_t[:2] + priors_cxcy_t[2:] / 2.0], axis=0)
    labels_col = labels.astype(jnp.float32).reshape(B, O, 1)
    locs_t = jnp.transpose(odm_locs, (0, 2, 1))

    loc_s, npos, focal_s = pl.pallas_call(
        _fused_kernel,
        grid=(B,),
        in_specs=[
            pl.BlockSpec((1, O, 4), lambda i: (i, 0, 0)),
            pl.BlockSpec((1, O, 1), lambda i: (i, 0, 0)),
            pl.BlockSpec((4, P), lambda i: (0, 0)),
            pl.BlockSpec((4, P), lambda i: (0, 0)),
            pl.BlockSpec((1, 4, P), lambda i: (i, 0, 0)),
            pl.BlockSpec((1, P, C), lambda i: (i, 0, 0)),
        ],
        out_specs=[
            pl.BlockSpec((1, 1, 1), lambda i: (i, 0, 0)),
            pl.BlockSpec((1, 1, 1), lambda i: (i, 0, 0)),
            pl.BlockSpec((1, 1, 1), lambda i: (i, 0, 0)),
        ],
        out_shape=[
            jax.ShapeDtypeStruct((B, 1, 1), jnp.float32),
            jax.ShapeDtypeStruct((B, 1, 1), jnp.float32),
            jax.ShapeDtypeStruct((B, 1, 1), jnp.float32),
        ],
        interpret=interpret,
    )(boxes, labels_col, priors_xy_t, priors_cxcy_t, locs_t, odm_scores)

    n_pos = jnp.sum(npos)
    return (jnp.sum(focal_s) + jnp.sum(loc_s)) / n_pos


def kernel(odm_locs, odm_scores, boxes, labels, priors_cxcy):
    return _run(odm_locs, odm_scores, boxes, labels, priors_cxcy)


# speedup vs baseline: 1.1645x; 1.0144x over previous
import functools

import numpy as np
import jax
import jax.numpy as jnp
from jax.experimental import pallas as pl

N_PRIORS = 20000
N_CLASSES = 80
THRESHOLD = 0.7
BIG_I32 = np.int32(2**30)

_ATAN_COEFFS = (0.9999993329, -0.3332985605, 0.1994653599, -0.1390853351,
                0.0964200441, -0.0559098861, 0.0218612288, -0.0040454960)


def _arctan(x):
    ax = jnp.abs(x)
    inv = ax > 1.0
    t = jnp.where(inv, 1.0 / jnp.maximum(ax, 1e-30), ax)
    t2 = t * t
    poly = _ATAN_COEFFS[-1]
    for c in _ATAN_COEFFS[-2::-1]:
        poly = poly * t2 + c
    poly = poly * t
    r = jnp.where(inv, np.float32(np.pi / 2) - poly, poly)
    return jnp.where(x < 0, -r, r)


def _fused_kernel(boxes_ref, labels_ref, pxy_ref, pcxcy_ref, locs_ref,
                  scores_ref, loc_ref, npos_ref, focal_ref):
    boxes = boxes_ref[0]
    bx0 = boxes[:, 0:1]
    by0 = boxes[:, 1:2]
    bx1 = boxes[:, 2:3]
    by1 = boxes[:, 3:4]
    px0 = pxy_ref[0:1, :]
    py0 = pxy_ref[1:2, :]
    px1 = pxy_ref[2:3, :]
    py1 = pxy_ref[3:4, :]

    iw0 = jnp.clip(jnp.minimum(bx1, px1) - jnp.maximum(bx0, px0), 0.0, None)
    ih0 = jnp.clip(jnp.minimum(by1, py1) - jnp.maximum(by0, py0), 0.0, None)
    inter0 = iw0 * ih0
    a1 = (bx1 - bx0) * (by1 - by0)
    a2 = (px1 - px0) * (py1 - py0)
    overlap = inter0 / (a1 + a2 - inter0)

    n_obj = overlap.shape[0]
    o_iota = jax.lax.broadcasted_iota(jnp.int32, (n_obj, 1), 0)
    p_iota = jax.lax.broadcasted_iota(jnp.int32, (1, overlap.shape[1]), 1)

    ofp = jnp.max(overlap, axis=0, keepdims=True)

    ofo = jnp.max(overlap, axis=1, keepdims=True)
    pfo = jnp.min(
        jnp.where(overlap == ofo, p_iota, BIG_I32), axis=1, keepdims=True)
    valid = ofo > 0.0

    match = (pfo == p_iota) & valid
    k = jnp.where(match, 64 + o_iota,
                  jnp.where(overlap == ofp, 63 - o_iota, np.int32(-1)))
    m = jnp.max(k, axis=0, keepdims=True)
    has = m >= 64
    obj2 = jnp.where(has, m - 64, 63 - m)
    ofp2 = jnp.where(has, 1.0, ofp)

    onehot = (obj2 == o_iota).astype(jnp.float32)
    vals = jnp.concatenate([labels_ref[0], bx0, by0, bx1, by1], axis=1)
    gathered = jax.lax.dot_general(
        vals, onehot, (((0,), (0,)), ((), ())),
        preferred_element_type=jnp.float32)
    lab = jnp.where(ofp2 < THRESHOLD, -1.0, gathered[0:1, :])
    lab = jnp.where(ofp2 < THRESHOLD - 0.25, 0.0, lab)

    locs = locs_ref[0]
    pri = pcxcy_ref[...]
    cx = locs[0:1, :] * pri[2:3, :] / 10.0 + pri[0:1, :]
    cy = locs[1:2, :] * pri[3:4, :] / 10.0 + pri[1:2, :]
    w1 = jnp.exp(locs[2:3, :] / 5.0) * pri[2:3, :]
    h1 = jnp.exp(locs[3:4, :] / 5.0) * pri[3:4, :]
    b1x0 = cx - w1 / 2.0
    b1y0 = cy - h1 / 2.0
    b1x1 = cx + w1 / 2.0
    b1y1 = cy + h1 / 2.0

    b2x0 = gathered[1:2, :]
    b2y0 = gathered[2:3, :]
    b2x1 = gathered[3:4, :]
    b2y1 = gathered[4:5, :]
    w2 = b2x1 - b2x0
    h2 = b2y1 - b2y0

    iw = jnp.clip(jnp.minimum(b1x1, b2x1) - jnp.maximum(b1x0, b2x0),
                  0.0, None)
    ih = jnp.clip(jnp.minimum(b1y1, b2y1) - jnp.maximum(b1y0, b2y0),
                  0.0, None)
    inter = iw * ih
    union = w1 * h1 + w2 * h2 - inter
    iou = inter / (union + 1e-7)
    ew = jnp.clip(jnp.maximum(b1x1, b2x1) - jnp.minimum(b1x0, b2x0),
                  0.0, None)
    eh = jnp.clip(jnp.maximum(b1y1, b2y1) - jnp.minimum(b1y0, b2y0),
                  0.0, None)
    cdiag = ew * ew + eh * eh + 1e-7
    c2x = (b2x0 + b2x1) / 2.0
    c2y = (b2y0 + b2y1) / 2.0
    rho2 = (cx - c2x) ** 2 + (cy - c2y) ** 2
    u = rho2 / cdiag
    v = (4.0 / (np.pi ** 2)) * (
        _arctan(w2 / (h2 + 1e-7)) - _arctan(w1 / (h1 + 1e-7))
    ) ** 2
    alpha = v / ((1.0 - iou) + v + 1e-7)
    ciou = iou - (u + alpha * v)

    posf = (lab > 0.0).astype(jnp.float32)
    loc_ref[0] = jnp.sum(posf * (1.0 - ciou), keepdims=True)
    npos_ref[0] = jnp.sum(posf, keepdims=True)

    x = scores_ref[0]
    t = lab.reshape((x.shape[0], 1))
    cr = jax.lax.broadcasted_iota(jnp.int32, (1, x.shape[1]), 1).astype(
        jnp.float32) + 1.0
    un = jnp.exp(-x)
    w = 1.0 + un
    r = 1.0 / w
    s = jnp.log(w)
    t2n = (r * r) * (x + s)
    xsel = jnp.where(t == cr, x, 0.0)
    ones_c = jnp.ones((x.shape[1], 1), dtype=jnp.float32)
    s2 = jax.lax.dot_general(
        t2n, ones_c, (((1,), (0,)), ((), ())),
        preferred_element_type=jnp.float32)
    xs = jax.lax.dot_general(
        xsel, ones_c, (((1,), (0,)), ((), ())),
        preferred_element_type=jnp.float32)
    row_ok = (t >= 0.0).astype(jnp.float32)
    neg_all = jnp.sum(row_ok * s2, keepdims=True)
    us = jnp.exp(-xs)
    ps = 1.0 / (1.0 + us)
    ss = jnp.log(1.0 + us)
    t1n_s = (us * ps) ** 2 * ss
    t2n_s = ps * ps * (xs + ss)
    pos_row = (t > 0.0).astype(jnp.float32)
    corr = jnp.sum(pos_row * (0.25 * t1n_s - 0.75 * t2n_s), keepdims=True)
    focal_ref[0] = 0.75 * neg_all + corr


@functools.partial(jax.jit, static_argnames=("interpret",))
def _run(odm_locs, odm_scores, boxes, labels, priors_cxcy, interpret=False):
    B, P, C = odm_scores.shape
    O = boxes.shape[1]
    priors_cxcy_t = priors_cxcy.T
    priors_xy_t = jnp.concatenate(
        [priors_cxcy_t[:2] - priors_cxcy_t[2:] / 2.0,
         priors_cxcy_t[:2] + priors_cxcy_t[2:] / 2.0], axis=0)
    labels_col = labels.astype(jnp.float32).reshape(B, O, 1)
    locs_t = jnp.transpose(odm_locs, (0, 2, 1))

    loc_s, npos, focal_s = pl.pallas_call(
        _fused_kernel,
        grid=(B,),
        in_specs=[
            pl.BlockSpec((1, O, 4), lambda i: (i, 0, 0)),
            pl.BlockSpec((1, O, 1), lambda i: (i, 0, 0)),
            pl.BlockSpec((4, P), lambda i: (0, 0)),
            pl.BlockSpec((4, P), lambda i: (0, 0)),
            pl.BlockSpec((1, 4, P), lambda i: (i, 0, 0)),
            pl.BlockSpec((1, P, C), lambda i: (i, 0, 0)),
        ],
        out_specs=[
            pl.BlockSpec((1, 1, 1), lambda i: (i, 0, 0)),
            pl.BlockSpec((1, 1, 1), lambda i: (i, 0, 0)),
            pl.BlockSpec((1, 1, 1), lambda i: (i, 0, 0)),
        ],
        out_shape=[
            jax.ShapeDtypeStruct((B, 1, 1), jnp.float32),
            jax.ShapeDtypeStruct((B, 1, 1), jnp.float32),
            jax.ShapeDtypeStruct((B, 1, 1), jnp.float32),
        ],
        interpret=interpret,
    )(boxes, labels_col, priors_xy_t, priors_cxcy_t, locs_t, odm_scores)

    n_pos = jnp.sum(npos)
    return (jnp.sum(focal_s) + jnp.sum(loc_s)) / n_pos


def kernel(odm_locs, odm_scores, boxes, labels, priors_cxcy):
    return _run(odm_locs, odm_scores, boxes, labels, priors_cxcy)
